# baseline (speedup 1.0000x reference)
import functools

import jax
import jax.numpy as jnp
from jax import lax
from jax.experimental import pallas as pl
from jax.experimental.pallas import tpu as pltpu
from jax.experimental.pallas import tpu_sc as plsc

C = 1000
A = 64
D = 512
K = 16
B = 4096
SCALE = 0.5

NC = 2
NS = 16
NW = NC * NS
BPW = B // NW
RPW = BPW * K
CHUNK_B = 4
CHUNK_R = CHUNK_B * K
N_CHUNKS = BPW // CHUNK_B
NBUF = 3

_mesh = plsc.VectorSubcoreMesh(core_axis_name="c", subcore_axis_name="s")


def _worker_id():
    return lax.axis_index("s") * NC + lax.axis_index("c")


def _lane_broadcast(vec, k):
    idx = jnp.full((16, 1), k, jnp.int32)
    dnums = lax.GatherDimensionNumbers(
        offset_dims=(), collapsed_slice_dims=(0,), start_index_map=(0,))
    return lax.gather(vec, idx, dnums, (1,),
                      mode=lax.GatherScatterMode.PROMISE_IN_BOUNDS)


CNT_W = 128


@functools.partial(
    pl.kernel,
    mesh=_mesh,
    out_type=(
        jax.ShapeDtypeStruct((B, CNT_W), jnp.float32),
        jax.ShapeDtypeStruct((B, D), jnp.float32),
    ),
    scratch_types=[
        pltpu.VMEM((BPW,), jnp.int32),
        pltpu.VMEM((BPW, CNT_W), jnp.float32),
        pltpu.VMEM((BPW, D), jnp.float32),
        pltpu.SemaphoreType.DMA,
    ],
)
def _sc_gather_cnt_base(labels_hbm, counts_hbm, base_hbm, cnt_out, base_out,
                        idx_v, cnt_v, base_v, sem):
    wid = _worker_id()
    b0 = wid * BPW
    pltpu.sync_copy(labels_hbm.at[pl.ds(b0, BPW)], idx_v)
    cp1 = pltpu.async_copy(counts_hbm.at[idx_v], cnt_v, sem)
    cp2 = pltpu.async_copy(base_hbm.at[idx_v], base_v, sem)
    cp1.wait()
    cp2.wait()
    pltpu.sync_copy(cnt_v, cnt_out.at[pl.ds(b0, BPW)])
    pltpu.sync_copy(base_v, base_out.at[pl.ds(b0, BPW)])


_TC_BLK = 256


def _tc_topk_body(cnt_ref, g_ref, ids_ref):
    cnt = cnt_ref[:, :A]
    g = g_ref[...]
    norm = jnp.sqrt(jnp.sum(cnt * cnt, axis=-1, keepdims=True) + 1e-12)
    w = 1.0 - cnt / jnp.maximum(norm, 1e-12)
    s = jnp.log(jnp.maximum(w, 1e-12)) + g
    iota = lax.broadcasted_iota(jnp.int32, (_TC_BLK, A), 1)
    cols = []
    for _ in range(K):
        idx = jnp.argmax(s, axis=1)[:, None]
        cols.append(idx)
        s = jnp.where(iota == idx, -jnp.inf, s)
    ids_ref[...] = jnp.concatenate(cols, axis=1)


def _tc_topk(cnt, g):
    grid = B // _TC_BLK
    return pl.pallas_call(
        _tc_topk_body,
        grid=(grid,),
        in_specs=[
            pl.BlockSpec((_TC_BLK, CNT_W), lambda i: (i, 0)),
            pl.BlockSpec((_TC_BLK, A), lambda i: (i, 0)),
        ],
        out_specs=pl.BlockSpec((_TC_BLK, K), lambda i: (i, 0)),
        out_shape=jax.ShapeDtypeStruct((B, K), jnp.int32),
    )(cnt, g)


@functools.partial(
    pl.kernel,
    mesh=_mesh,
    out_type=jax.ShapeDtypeStruct((B * K, D), jnp.float32),
    scratch_types=[
        pltpu.VMEM((RPW,), jnp.int32),
        pltpu.VMEM((RPW,), jnp.int32),
        pltpu.VMEM((RPW,), jnp.float32),
        pltpu.VMEM((N_CHUNKS, CHUNK_R), jnp.int32),
        pltpu.VMEM((CHUNK_R, D), jnp.float32),
        pltpu.VMEM((CHUNK_R, D), jnp.float32),
        pltpu.VMEM((CHUNK_R, D), jnp.float32),
        pltpu.VMEM((CHUNK_B, D), jnp.float32),
        pltpu.VMEM((CHUNK_B, D), jnp.float32),
        pltpu.VMEM((CHUNK_B, D), jnp.float32),
        pltpu.SemaphoreType.DMA,
        pltpu.SemaphoreType.DMA,
        pltpu.SemaphoreType.DMA,
        pltpu.SemaphoreType.DMA,
        pltpu.SemaphoreType.DMA,
        pltpu.SemaphoreType.DMA,
    ],
)
def _sc_gather_mix(attr_hbm, ids_hbm, tgt_hbm, img_hbm, alpha_hbm, out_hbm,
                   ids_v, tgt_v, alp_v, idx2d, rows0, rows1, rows2,
                   img0, img1, img2, sg0, sg1, sg2, ss0, ss1, ss2):
    wid = _worker_id()
    r0 = wid * RPW
    b0 = wid * BPW

    pltpu.sync_copy(ids_hbm.at[pl.ds(r0, RPW)], ids_v)
    pltpu.sync_copy(tgt_hbm.at[pl.ds(r0, RPW)], tgt_v)
    pltpu.sync_copy(alpha_hbm.at[pl.ds(r0, RPW)], alp_v)
    for c in range(N_CHUNKS):
        for v in range(CHUNK_R // 16):
            sl = pl.ds(c * CHUNK_R + 16 * v, 16)
            idx2d[c, pl.ds(16 * v, 16)] = tgt_v[sl] * A + ids_v[sl]

    rows = (rows0, rows1, rows2)
    imgs = (img0, img1, img2)
    sgs = (sg0, sg1, sg2)
    sss = (ss0, ss1, ss2)

    def start_in(c, p):
        pltpu.async_copy(attr_hbm.at[idx2d.at[c]], rows[p], sgs[p])
        pltpu.async_copy(img_hbm.at[pl.ds(b0 + c * CHUNK_B, CHUNK_B)],
                         imgs[p], sgs[p])

    def wait_in(c, p):
        pltpu.make_async_copy(attr_hbm.at[idx2d.at[c]], rows[p], sgs[p]).wait()
        pltpu.make_async_copy(img_hbm.at[pl.ds(b0 + c * CHUNK_B, CHUNK_B)],
                              imgs[p], sgs[p]).wait()

    def start_out(c, p):
        pltpu.async_copy(rows[p], out_hbm.at[pl.ds(r0 + c * CHUNK_R, CHUNK_R)],
                         sss[p])

    def wait_out(c, p):
        pltpu.make_async_copy(
            rows[p], out_hbm.at[pl.ds(r0 + c * CHUNK_R, CHUNK_R)],
            sss[p]).wait()

    def compute(c, p):
        rows_p = rows[p]
        img_p = imgs[p]
        for bl in range(CHUNK_B):
            av_vec = alp_v[pl.ds(c * CHUNK_R + bl * K, K)]
            avs = [_lane_broadcast(av_vec, k) for k in range(K)]

            def j_body(j, carry, bl=bl, avs=avs):
                sl = pl.ds(16 * j, 16)
                iv = img_p[bl, sl]
                for k in range(K):
                    r = bl * K + k
                    rows_p[r, sl] = iv + avs[k] * rows_p[r, sl]
                return carry

            lax.fori_loop(0, D // 16, j_body, 0)

    start_in(0, 0)
    N_MAIN = (N_CHUNKS // NBUF) * NBUF

    def tri_body(i, carry):
        for p in range(NBUF):
            c = NBUF * i + p
            wait_in(c, p)
            if p < 2:
                @pl.when(i >= 1)
                def _():
                    wait_out(c - 2, (p + 1) % NBUF)
            else:
                wait_out(c - 2, p - 2)
            start_in(c + 1, (p + 1) % NBUF)
            compute(c, p)
            start_out(c, p)
        return carry

    lax.fori_loop(0, N_MAIN // NBUF, tri_body, 0)
    for c in range(N_MAIN, N_CHUNKS):
        p = c % NBUF
        wait_in(c, p)
        wait_out(c - 2, (c - 2) % NBUF)
        if c + 1 < N_CHUNKS:
            start_in(c + 1, (c + 1) % NBUF)
        compute(c, p)
        start_out(c, p)
    wait_out(N_CHUNKS - 2, (N_CHUNKS - 2) % NBUF)
    wait_out(N_CHUNKS - 1, (N_CHUNKS - 1) % NBUF)


def kernel(labels, img_feats, attr_feats, base_feats, counts):
    key = jax.random.key(42)
    g = jax.random.gumbel(key, (B, A))
    alpha = jnp.maximum(
        jax.random.uniform(jax.random.fold_in(key, 1), (B, K, 1)), SCALE)
    alpha_flat = alpha.reshape(B * K)

    labels_i32 = labels.astype(jnp.int32)
    aug_targets = jnp.repeat(labels, K)

    counts_p = jnp.pad(counts, ((0, 0), (0, CNT_W - A)))
    cnt_g, base_feat = _sc_gather_cnt_base(labels_i32, counts_p, base_feats)
    ids = _tc_topk(cnt_g, g)

    attr_flat = attr_feats.reshape(C * A, D)
    diff_feats = _sc_gather_mix(
        attr_flat,
        ids.reshape(B * K),
        aug_targets.astype(jnp.int32),
        img_feats,
        alpha_flat,
    )
    return base_feat, diff_feats, aug_targets


# speedup vs baseline: 1.0382x; 1.0382x over previous
import functools

import jax
import jax.numpy as jnp
from jax import lax
from jax.experimental import pallas as pl
from jax.experimental.pallas import tpu as pltpu
from jax.experimental.pallas import tpu_sc as plsc

C = 1000
A = 64
D = 512
K = 16
B = 4096
SCALE = 0.5

NC = 2
NS = 16
NW = NC * NS
BPW = B // NW
RPW = BPW * K
CHUNK_B = 2
CHUNK_R = CHUNK_B * K
N_CHUNKS = BPW // CHUNK_B
NBUF = 6
DEPTH = 3

_mesh = plsc.VectorSubcoreMesh(core_axis_name="c", subcore_axis_name="s")


def _worker_id():
    return lax.axis_index("s") * NC + lax.axis_index("c")


def _lane_broadcast(vec, k):
    idx = jnp.full((16, 1), k, jnp.int32)
    dnums = lax.GatherDimensionNumbers(
        offset_dims=(), collapsed_slice_dims=(0,), start_index_map=(0,))
    return lax.gather(vec, idx, dnums, (1,),
                      mode=lax.GatherScatterMode.PROMISE_IN_BOUNDS)


CNT_W = 128


@functools.partial(
    pl.kernel,
    mesh=_mesh,
    out_type=(
        jax.ShapeDtypeStruct((B, CNT_W), jnp.float32),
        jax.ShapeDtypeStruct((B, D), jnp.float32),
    ),
    scratch_types=[
        pltpu.VMEM((BPW,), jnp.int32),
        pltpu.VMEM((BPW, CNT_W), jnp.float32),
        pltpu.VMEM((BPW, D), jnp.float32),
        pltpu.SemaphoreType.DMA,
    ],
)
def _sc_gather_cnt_base(labels_hbm, counts_hbm, base_hbm, cnt_out, base_out,
                        idx_v, cnt_v, base_v, sem):
    wid = _worker_id()
    b0 = wid * BPW
    pltpu.sync_copy(labels_hbm.at[pl.ds(b0, BPW)], idx_v)
    cp1 = pltpu.async_copy(counts_hbm.at[idx_v], cnt_v, sem)
    cp2 = pltpu.async_copy(base_hbm.at[idx_v], base_v, sem)
    cp1.wait()
    cp2.wait()
    pltpu.sync_copy(cnt_v, cnt_out.at[pl.ds(b0, BPW)])
    pltpu.sync_copy(base_v, base_out.at[pl.ds(b0, BPW)])


_TC_BLK = 256


def _tc_topk_body(cnt_ref, g_ref, ids_ref):
    cnt = cnt_ref[:, :A]
    g = g_ref[...]
    norm = jnp.sqrt(jnp.sum(cnt * cnt, axis=-1, keepdims=True) + 1e-12)
    w = 1.0 - cnt / jnp.maximum(norm, 1e-12)
    s = jnp.log(jnp.maximum(w, 1e-12)) + g
    iota = lax.broadcasted_iota(jnp.int32, (_TC_BLK, A), 1)
    cols = []
    for _ in range(K):
        idx = jnp.argmax(s, axis=1)[:, None]
        cols.append(idx)
        s = jnp.where(iota == idx, -jnp.inf, s)
    ids_ref[...] = jnp.concatenate(cols, axis=1)


def _tc_topk(cnt, g):
    grid = B // _TC_BLK
    return pl.pallas_call(
        _tc_topk_body,
        grid=(grid,),
        in_specs=[
            pl.BlockSpec((_TC_BLK, CNT_W), lambda i: (i, 0)),
            pl.BlockSpec((_TC_BLK, A), lambda i: (i, 0)),
        ],
        out_specs=pl.BlockSpec((_TC_BLK, K), lambda i: (i, 0)),
        out_shape=jax.ShapeDtypeStruct((B, K), jnp.int32),
    )(cnt, g)


@functools.partial(
    pl.kernel,
    mesh=_mesh,
    out_type=jax.ShapeDtypeStruct((B * K, D), jnp.float32),
    scratch_types=[
        pltpu.VMEM((RPW,), jnp.int32),
        pltpu.VMEM((RPW,), jnp.int32),
        pltpu.VMEM((RPW,), jnp.float32),
        pltpu.VMEM((N_CHUNKS, CHUNK_R), jnp.int32),
    ] + [pltpu.VMEM((CHUNK_R, D), jnp.float32)] * NBUF
      + [pltpu.VMEM((CHUNK_B, D), jnp.float32)] * NBUF
      + [pltpu.SemaphoreType.DMA] * (2 * NBUF),
)
def _sc_gather_mix(attr_hbm, ids_hbm, tgt_hbm, img_hbm, alpha_hbm, out_hbm,
                   ids_v, tgt_v, alp_v, idx2d, *bufs):
    rows = bufs[0:NBUF]
    imgs = bufs[NBUF:2 * NBUF]
    sgs = bufs[2 * NBUF:3 * NBUF]
    sss = bufs[3 * NBUF:4 * NBUF]
    wid = _worker_id()
    r0 = wid * RPW
    b0 = wid * BPW

    pltpu.sync_copy(ids_hbm.at[pl.ds(r0, RPW)], ids_v)
    pltpu.sync_copy(tgt_hbm.at[pl.ds(r0, RPW)], tgt_v)
    pltpu.sync_copy(alpha_hbm.at[pl.ds(r0, RPW)], alp_v)
    for c in range(N_CHUNKS):
        for v in range(CHUNK_R // 16):
            sl = pl.ds(c * CHUNK_R + 16 * v, 16)
            idx2d[c, pl.ds(16 * v, 16)] = tgt_v[sl] * A + ids_v[sl]

    def start_in(c, p):
        pltpu.async_copy(attr_hbm.at[idx2d.at[c]], rows[p], sgs[p])
        pltpu.async_copy(img_hbm.at[pl.ds(b0 + c * CHUNK_B, CHUNK_B)],
                         imgs[p], sgs[p])

    def wait_in(c, p):
        pltpu.make_async_copy(attr_hbm.at[idx2d.at[c]], rows[p], sgs[p]).wait()
        pltpu.make_async_copy(img_hbm.at[pl.ds(b0 + c * CHUNK_B, CHUNK_B)],
                              imgs[p], sgs[p]).wait()

    def start_out(c, p):
        pltpu.async_copy(rows[p], out_hbm.at[pl.ds(r0 + c * CHUNK_R, CHUNK_R)],
                         sss[p])

    def wait_out(c, p):
        pltpu.make_async_copy(
            rows[p], out_hbm.at[pl.ds(r0 + c * CHUNK_R, CHUNK_R)],
            sss[p]).wait()

    def compute(c, p):
        rows_p = rows[p]
        img_p = imgs[p]
        for bl in range(CHUNK_B):
            av_vec = alp_v[pl.ds(c * CHUNK_R + bl * K, K)]
            avs = [_lane_broadcast(av_vec, k) for k in range(K)]

            def j_body(j, carry, bl=bl, avs=avs):
                sl = pl.ds(16 * j, 16)
                iv = img_p[bl, sl]
                for k in range(K):
                    r = bl * K + k
                    rows_p[r, sl] = iv + avs[k] * rows_p[r, sl]
                return carry

            lax.fori_loop(0, D // 16, j_body, 0)

    LAG = NBUF - DEPTH
    for c0 in range(DEPTH):
        start_in(c0, c0)
    N_MAIN = (N_CHUNKS // NBUF) * NBUF

    def ring_body(i, carry):
        for p in range(NBUF):
            c = NBUF * i + p
            wait_in(c, p)
            if p < LAG:
                @pl.when(i >= 1)
                def _():
                    wait_out(c - LAG, (p + DEPTH) % NBUF)
            else:
                wait_out(c - LAG, p - LAG)
            start_in(c + DEPTH, (p + DEPTH) % NBUF)
            compute(c, p)
            start_out(c, p)
        return carry

    lax.fori_loop(0, N_MAIN // NBUF, ring_body, 0)
    for c in range(N_MAIN, N_CHUNKS):
        p = c % NBUF
        wait_in(c, p)
        wait_out(c - LAG, (c - LAG) % NBUF)
        if c + DEPTH < N_CHUNKS:
            start_in(c + DEPTH, (c + DEPTH) % NBUF)
        compute(c, p)
        start_out(c, p)
    for c in range(N_CHUNKS - LAG, N_CHUNKS):
        wait_out(c, c % NBUF)


def kernel(labels, img_feats, attr_feats, base_feats, counts):
    key = jax.random.key(42)
    g = jax.random.gumbel(key, (B, A))
    alpha = jnp.maximum(
        jax.random.uniform(jax.random.fold_in(key, 1), (B, K, 1)), SCALE)
    alpha_flat = alpha.reshape(B * K)

    labels_i32 = labels.astype(jnp.int32)
    aug_targets = jnp.repeat(labels, K)

    counts_p = jnp.pad(counts, ((0, 0), (0, CNT_W - A)))
    cnt_g, base_feat = _sc_gather_cnt_base(labels_i32, counts_p, base_feats)
    ids = _tc_topk(cnt_g, g)

    attr_flat = attr_feats.reshape(C * A, D)
    diff_feats = _sc_gather_mix(
        attr_flat,
        ids.reshape(B * K),
        aug_targets.astype(jnp.int32),
        img_feats,
        alpha_flat,
    )
    return base_feat, diff_feats, aug_targets


# speedup vs baseline: 1.0437x; 1.0053x over previous
import functools

import jax
import jax.numpy as jnp
from jax import lax
from jax.experimental import pallas as pl
from jax.experimental.pallas import tpu as pltpu
from jax.experimental.pallas import tpu_sc as plsc

C = 1000
A = 64
D = 512
K = 16
B = 4096
SCALE = 0.5

NC = 2
NS = 16
NW = NC * NS
BPW = B // NW
RPW = BPW * K
CHUNK_B = 2
CHUNK_R = CHUNK_B * K
N_CHUNKS = BPW // CHUNK_B
NBUF = 4
DEPTH = 2

_mesh = plsc.VectorSubcoreMesh(core_axis_name="c", subcore_axis_name="s")


def _worker_id():
    return lax.axis_index("s") * NC + lax.axis_index("c")


def _lane_broadcast(vec, k):
    idx = jnp.full((16, 1), k, jnp.int32)
    dnums = lax.GatherDimensionNumbers(
        offset_dims=(), collapsed_slice_dims=(0,), start_index_map=(0,))
    return lax.gather(vec, idx, dnums, (1,),
                      mode=lax.GatherScatterMode.PROMISE_IN_BOUNDS)


CNT_W = 128


@functools.partial(
    pl.kernel,
    mesh=_mesh,
    out_type=(
        jax.ShapeDtypeStruct((B, CNT_W), jnp.float32),
        jax.ShapeDtypeStruct((B, D), jnp.float32),
    ),
    scratch_types=[
        pltpu.VMEM((BPW,), jnp.int32),
        pltpu.VMEM((BPW, CNT_W), jnp.float32),
        pltpu.VMEM((BPW, D), jnp.float32),
        pltpu.SemaphoreType.DMA,
    ],
)
def _sc_gather_cnt_base(labels_hbm, counts_hbm, base_hbm, cnt_out, base_out,
                        idx_v, cnt_v, base_v, sem):
    wid = _worker_id()
    b0 = wid * BPW
    pltpu.sync_copy(labels_hbm.at[pl.ds(b0, BPW)], idx_v)
    cp1 = pltpu.async_copy(counts_hbm.at[idx_v], cnt_v, sem)
    cp2 = pltpu.async_copy(base_hbm.at[idx_v], base_v, sem)
    cp1.wait()
    cp2.wait()
    pltpu.sync_copy(cnt_v, cnt_out.at[pl.ds(b0, BPW)])
    pltpu.sync_copy(base_v, base_out.at[pl.ds(b0, BPW)])


_TC_BLK = 256


def _tc_topk_body(cnt_ref, g_ref, ids_ref):
    cnt = cnt_ref[:, :A]
    g = g_ref[...]
    norm = jnp.sqrt(jnp.sum(cnt * cnt, axis=-1, keepdims=True) + 1e-12)
    w = 1.0 - cnt / jnp.maximum(norm, 1e-12)
    s = jnp.log(jnp.maximum(w, 1e-12)) + g
    iota = lax.broadcasted_iota(jnp.int32, (_TC_BLK, A), 1)
    cols = []
    for _ in range(K):
        idx = jnp.argmax(s, axis=1)[:, None]
        cols.append(idx)
        s = jnp.where(iota == idx, -jnp.inf, s)
    ids_ref[...] = jnp.concatenate(cols, axis=1)


def _tc_topk(cnt, g):
    grid = B // _TC_BLK
    return pl.pallas_call(
        _tc_topk_body,
        grid=(grid,),
        in_specs=[
            pl.BlockSpec((_TC_BLK, CNT_W), lambda i: (i, 0)),
            pl.BlockSpec((_TC_BLK, A), lambda i: (i, 0)),
        ],
        out_specs=pl.BlockSpec((_TC_BLK, K), lambda i: (i, 0)),
        out_shape=jax.ShapeDtypeStruct((B, K), jnp.int32),
    )(cnt, g)


@functools.partial(
    pl.kernel,
    mesh=_mesh,
    out_type=jax.ShapeDtypeStruct((B * K, D), jnp.float32),
    scratch_types=[
        pltpu.VMEM((RPW,), jnp.int32),
        pltpu.VMEM((RPW,), jnp.int32),
        pltpu.VMEM((RPW,), jnp.float32),
        pltpu.VMEM((N_CHUNKS, CHUNK_R), jnp.int32),
    ] + [pltpu.VMEM((CHUNK_R, D), jnp.float32)] * NBUF
      + [pltpu.VMEM((CHUNK_B, D), jnp.float32)] * NBUF
      + [pltpu.SemaphoreType.DMA] * (2 * NBUF),
)
def _sc_gather_mix(attr_hbm, ids_hbm, tgt_hbm, img_hbm, alpha_hbm, out_hbm,
                   ids_v, tgt_v, alp_v, idx2d, *bufs):
    rows = bufs[0:NBUF]
    imgs = bufs[NBUF:2 * NBUF]
    sgs = bufs[2 * NBUF:3 * NBUF]
    sss = bufs[3 * NBUF:4 * NBUF]
    wid = _worker_id()
    r0 = wid * RPW
    b0 = wid * BPW

    pltpu.sync_copy(ids_hbm.at[pl.ds(r0, RPW)], ids_v)
    pltpu.sync_copy(tgt_hbm.at[pl.ds(r0, RPW)], tgt_v)
    pltpu.sync_copy(alpha_hbm.at[pl.ds(r0, RPW)], alp_v)
    for c in range(N_CHUNKS):
        for v in range(CHUNK_R // 16):
            sl = pl.ds(c * CHUNK_R + 16 * v, 16)
            idx2d[c, pl.ds(16 * v, 16)] = tgt_v[sl] * A + ids_v[sl]

    def start_in(c, p):
        pltpu.async_copy(attr_hbm.at[idx2d.at[c]], rows[p], sgs[p])
        pltpu.async_copy(img_hbm.at[pl.ds(b0 + c * CHUNK_B, CHUNK_B)],
                         imgs[p], sgs[p])

    def wait_in(c, p):
        pltpu.make_async_copy(attr_hbm.at[idx2d.at[c]], rows[p], sgs[p]).wait()
        pltpu.make_async_copy(img_hbm.at[pl.ds(b0 + c * CHUNK_B, CHUNK_B)],
                              imgs[p], sgs[p]).wait()

    def start_out(c, p):
        pltpu.async_copy(rows[p], out_hbm.at[pl.ds(r0 + c * CHUNK_R, CHUNK_R)],
                         sss[p])

    def wait_out(c, p):
        pltpu.make_async_copy(
            rows[p], out_hbm.at[pl.ds(r0 + c * CHUNK_R, CHUNK_R)],
            sss[p]).wait()

    def compute(c, p):
        rows_p = rows[p]
        img_p = imgs[p]
        for bl in range(CHUNK_B):
            av_vec = alp_v[pl.ds(c * CHUNK_R + bl * K, K)]
            avs = [_lane_broadcast(av_vec, k) for k in range(K)]

            def j_body(j, carry, bl=bl, avs=avs):
                sl = pl.ds(16 * j, 16)
                iv = img_p[bl, sl]
                for k in range(K):
                    r = bl * K + k
                    rows_p[r, sl] = iv + avs[k] * rows_p[r, sl]
                return carry

            lax.fori_loop(0, D // 16, j_body, 0)

    LAG = NBUF - DEPTH
    for c0 in range(DEPTH):
        start_in(c0, c0)
    N_MAIN = (N_CHUNKS // NBUF) * NBUF

    def ring_body(i, carry):
        for p in range(NBUF):
            c = NBUF * i + p
            wait_in(c, p)
            if p < LAG:
                @pl.when(i >= 1)
                def _():
                    wait_out(c - LAG, (p + DEPTH) % NBUF)
            else:
                wait_out(c - LAG, p - LAG)
            if NBUF * (N_MAIN // NBUF - 1) + p + DEPTH < N_CHUNKS:
                start_in(c + DEPTH, (p + DEPTH) % NBUF)
            else:
                @pl.when(c + DEPTH < N_CHUNKS)
                def _():
                    start_in(c + DEPTH, (p + DEPTH) % NBUF)
            compute(c, p)
            start_out(c, p)
        return carry

    lax.fori_loop(0, N_MAIN // NBUF, ring_body, 0)
    for c in range(N_MAIN, N_CHUNKS):
        p = c % NBUF
        wait_in(c, p)
        wait_out(c - LAG, (c - LAG) % NBUF)
        if c + DEPTH < N_CHUNKS:
            start_in(c + DEPTH, (c + DEPTH) % NBUF)
        compute(c, p)
        start_out(c, p)
    for c in range(N_CHUNKS - LAG, N_CHUNKS):
        wait_out(c, c % NBUF)


def kernel(labels, img_feats, attr_feats, base_feats, counts):
    key = jax.random.key(42)
    g = jax.random.gumbel(key, (B, A))
    alpha = jnp.maximum(
        jax.random.uniform(jax.random.fold_in(key, 1), (B, K, 1)), SCALE)
    alpha_flat = alpha.reshape(B * K)

    labels_i32 = labels.astype(jnp.int32)
    aug_targets = jnp.repeat(labels, K)

    counts_p = jnp.pad(counts, ((0, 0), (0, CNT_W - A)))
    cnt_g, base_feat = _sc_gather_cnt_base(labels_i32, counts_p, base_feats)
    ids = _tc_topk(cnt_g, g)

    attr_flat = attr_feats.reshape(C * A, D)
    diff_feats = _sc_gather_mix(
        attr_flat,
        ids.reshape(B * K),
        aug_targets.astype(jnp.int32),
        img_feats,
        alpha_flat,
    )
    return base_feat, diff_feats, aug_targets


# speedup vs baseline: 1.1668x; 1.1179x over previous
import functools

import jax
import jax.numpy as jnp
from jax import lax
from jax.experimental import pallas as pl
from jax.experimental.pallas import tpu as pltpu
from jax.experimental.pallas import tpu_sc as plsc

C = 1000
A = 64
D = 512
K = 16
B = 4096
SCALE = 0.5

NC = 2
NS = 16
NW = NC * NS
BPW = B // NW
RPW = BPW * K
CHUNK_B = 2
CHUNK_R = CHUNK_B * K
N_CHUNKS = BPW // CHUNK_B
NBUF = 4
DEPTH = 2

_mesh = plsc.VectorSubcoreMesh(core_axis_name="c", subcore_axis_name="s")


def _worker_id():
    return lax.axis_index("s") * NC + lax.axis_index("c")


def _lane_broadcast(vec, k):
    idx = jnp.full((16, 1), k, jnp.int32)
    dnums = lax.GatherDimensionNumbers(
        offset_dims=(), collapsed_slice_dims=(0,), start_index_map=(0,))
    return lax.gather(vec, idx, dnums, (1,),
                      mode=lax.GatherScatterMode.PROMISE_IN_BOUNDS)


CNT_W = 128


@functools.partial(
    pl.kernel,
    mesh=_mesh,
    out_type=(
        jax.ShapeDtypeStruct((B, CNT_W), jnp.float32),
        jax.ShapeDtypeStruct((B, D), jnp.float32),
    ),
    scratch_types=[
        pltpu.VMEM((BPW,), jnp.int32),
        pltpu.VMEM((BPW, CNT_W), jnp.float32),
        pltpu.VMEM((BPW, D), jnp.float32),
        pltpu.SemaphoreType.DMA,
    ],
)
def _sc_gather_cnt_base(labels_hbm, counts_hbm, base_hbm, cnt_out, base_out,
                        idx_v, cnt_v, base_v, sem):
    wid = _worker_id()
    b0 = wid * BPW
    pltpu.sync_copy(labels_hbm.at[pl.ds(b0, BPW)], idx_v)
    cp1 = pltpu.async_copy(counts_hbm.at[idx_v], cnt_v, sem)
    cp2 = pltpu.async_copy(base_hbm.at[idx_v], base_v, sem)
    cp1.wait()
    cp2.wait()
    pltpu.sync_copy(cnt_v, cnt_out.at[pl.ds(b0, BPW)])
    pltpu.sync_copy(base_v, base_out.at[pl.ds(b0, BPW)])


_TC_BLK = 512


def _tc_topk_body(cnt_ref, g_ref, ids_ref):
    cnt = cnt_ref[:A, :]
    g = g_ref[...]
    norm = jnp.sqrt(jnp.sum(cnt * cnt, axis=0, keepdims=True) + 1e-12)
    w = 1.0 - cnt / jnp.maximum(norm, 1e-12)
    s = jnp.log(jnp.maximum(w, 1e-12)) + g
    iota = lax.broadcasted_iota(jnp.int32, (A, _TC_BLK), 0)
    rows = []
    for _ in range(K):
        idx = jnp.argmax(s, axis=0)[None, :]
        rows.append(idx)
        s = jnp.where(iota == idx, -jnp.inf, s)
    ids_ref[...] = jnp.concatenate(rows, axis=0)


def _tc_topk(cnt_t, g_t):
    grid = B // _TC_BLK
    return pl.pallas_call(
        _tc_topk_body,
        grid=(grid,),
        in_specs=[
            pl.BlockSpec((CNT_W, _TC_BLK), lambda i: (0, i)),
            pl.BlockSpec((A, _TC_BLK), lambda i: (0, i)),
        ],
        out_specs=pl.BlockSpec((K, _TC_BLK), lambda i: (0, i)),
        out_shape=jax.ShapeDtypeStruct((K, B), jnp.int32),
    )(cnt_t, g_t)


@functools.partial(
    pl.kernel,
    mesh=_mesh,
    out_type=jax.ShapeDtypeStruct((B * K, D), jnp.float32),
    scratch_types=[
        pltpu.VMEM((RPW,), jnp.int32),
        pltpu.VMEM((RPW,), jnp.int32),
        pltpu.VMEM((RPW,), jnp.float32),
        pltpu.VMEM((N_CHUNKS, CHUNK_R), jnp.int32),
    ] + [pltpu.VMEM((CHUNK_R, D), jnp.float32)] * NBUF
      + [pltpu.VMEM((CHUNK_B, D), jnp.float32)] * NBUF
      + [pltpu.SemaphoreType.DMA] * (2 * NBUF),
)
def _sc_gather_mix(attr_hbm, ids_hbm, tgt_hbm, img_hbm, alpha_hbm, out_hbm,
                   ids_v, tgt_v, alp_v, idx2d, *bufs):
    rows = bufs[0:NBUF]
    imgs = bufs[NBUF:2 * NBUF]
    sgs = bufs[2 * NBUF:3 * NBUF]
    sss = bufs[3 * NBUF:4 * NBUF]
    wid = _worker_id()
    r0 = wid * RPW
    b0 = wid * BPW

    pltpu.sync_copy(ids_hbm.at[pl.ds(r0, RPW)], ids_v)
    pltpu.sync_copy(tgt_hbm.at[pl.ds(r0, RPW)], tgt_v)
    pltpu.sync_copy(alpha_hbm.at[pl.ds(r0, RPW)], alp_v)
    for c in range(N_CHUNKS):
        for v in range(CHUNK_R // 16):
            sl = pl.ds(c * CHUNK_R + 16 * v, 16)
            idx2d[c, pl.ds(16 * v, 16)] = tgt_v[sl] * A + ids_v[sl]

    def start_in(c, p):
        pltpu.async_copy(attr_hbm.at[idx2d.at[c]], rows[p], sgs[p])
        pltpu.async_copy(img_hbm.at[pl.ds(b0 + c * CHUNK_B, CHUNK_B)],
                         imgs[p], sgs[p])

    def wait_in(c, p):
        pltpu.make_async_copy(attr_hbm.at[idx2d.at[c]], rows[p], sgs[p]).wait()
        pltpu.make_async_copy(img_hbm.at[pl.ds(b0 + c * CHUNK_B, CHUNK_B)],
                              imgs[p], sgs[p]).wait()

    def start_out(c, p):
        pltpu.async_copy(rows[p], out_hbm.at[pl.ds(r0 + c * CHUNK_R, CHUNK_R)],
                         sss[p])

    def wait_out(c, p):
        pltpu.make_async_copy(
            rows[p], out_hbm.at[pl.ds(r0 + c * CHUNK_R, CHUNK_R)],
            sss[p]).wait()

    def compute(c, p):
        rows_p = rows[p]
        img_p = imgs[p]
        for bl in range(CHUNK_B):
            av_vec = alp_v[pl.ds(c * CHUNK_R + bl * K, K)]
            avs = [_lane_broadcast(av_vec, k) for k in range(K)]

            def j_body(j, carry, bl=bl, avs=avs):
                sl = pl.ds(16 * j, 16)
                iv = img_p[bl, sl]
                for k in range(K):
                    r = bl * K + k
                    rows_p[r, sl] = iv + avs[k] * rows_p[r, sl]
                return carry

            lax.fori_loop(0, D // 16, j_body, 0)

    LAG = NBUF - DEPTH
    for c0 in range(DEPTH):
        start_in(c0, c0)
    N_MAIN = (N_CHUNKS // NBUF) * NBUF

    def ring_body(i, carry):
        for p in range(NBUF):
            c = NBUF * i + p
            wait_in(c, p)
            if p < LAG:
                @pl.when(i >= 1)
                def _():
                    wait_out(c - LAG, (p + DEPTH) % NBUF)
            else:
                wait_out(c - LAG, p - LAG)
            if NBUF * (N_MAIN // NBUF - 1) + p + DEPTH < N_CHUNKS:
                start_in(c + DEPTH, (p + DEPTH) % NBUF)
            else:
                @pl.when(c + DEPTH < N_CHUNKS)
                def _():
                    start_in(c + DEPTH, (p + DEPTH) % NBUF)
            compute(c, p)
            start_out(c, p)
        return carry

    lax.fori_loop(0, N_MAIN // NBUF, ring_body, 0)
    for c in range(N_MAIN, N_CHUNKS):
        p = c % NBUF
        wait_in(c, p)
        wait_out(c - LAG, (c - LAG) % NBUF)
        if c + DEPTH < N_CHUNKS:
            start_in(c + DEPTH, (c + DEPTH) % NBUF)
        compute(c, p)
        start_out(c, p)
    for c in range(N_CHUNKS - LAG, N_CHUNKS):
        wait_out(c, c % NBUF)


def kernel(labels, img_feats, attr_feats, base_feats, counts):
    key = jax.random.key(42)
    g = jax.random.gumbel(key, (B, A))
    alpha = jnp.maximum(
        jax.random.uniform(jax.random.fold_in(key, 1), (B, K, 1)), SCALE)
    alpha_flat = alpha.reshape(B * K)

    labels_i32 = labels.astype(jnp.int32)
    aug_targets = jnp.repeat(labels, K)

    counts_p = jnp.pad(counts, ((0, 0), (0, CNT_W - A)))
    cnt_g, base_feat = _sc_gather_cnt_base(labels_i32, counts_p, base_feats)
    ids_t = _tc_topk(cnt_g.T, g.T)

    attr_flat = attr_feats.reshape(C * A, D)
    diff_feats = _sc_gather_mix(
        attr_flat,
        ids_t.T.reshape(B * K),
        aug_targets.astype(jnp.int32),
        img_feats,
        alpha_flat,
    )
    return base_feat, diff_feats, aug_targets
